# initial kernel scaffold (unmeasured)
import jax
import jax.numpy as jnp
from jax import lax
from jax.experimental import pallas as pl
from jax.experimental.pallas import tpu as pltpu


def kernel(
    x,
):
    def body(*refs):
        pass

    out_shape = jax.ShapeDtypeStruct(..., jnp.float32)
    return pl.pallas_call(body, out_shape=out_shape)(...)


# baseline (device time: 103079 ns/iter reference)
import jax
import jax.numpy as jnp
from jax import lax
from jax.experimental import pallas as pl
from jax.experimental.pallas import tpu as pltpu

N_DEV = 16


def kernel(x):
    m, n = x.shape
    chunk = m // N_DEV

    def body(x_ref, out_ref, comm_ref, send_sems, recv_sems):
        my = lax.axis_index("i")
        left = (my - 1) % N_DEV
        right = (my + 1) % N_DEV

        barrier_sem = pltpu.get_barrier_semaphore()
        for nbr in [left, right]:
            pl.semaphore_signal(
                barrier_sem, inc=1,
                device_id=(nbr,), device_id_type=pl.DeviceIdType.MESH,
            )
        pl.semaphore_wait(barrier_sem, 2)

        comm_ref[0, :, :] = x_ref[pl.ds(my * chunk, chunk), :]
        for s in range(N_DEV - 1):
            send_slot = s % 2
            recv_slot = (s + 1) % 2
            rdma = pltpu.make_async_remote_copy(
                src_ref=comm_ref.at[send_slot],
                dst_ref=comm_ref.at[recv_slot],
                send_sem=send_sems.at[send_slot],
                recv_sem=recv_sems.at[recv_slot],
                device_id=(right,),
                device_id_type=pl.DeviceIdType.MESH,
            )
            rdma.start()
            rdma.wait()
            r = (my - 1 - s) % N_DEV
            comm_ref[recv_slot, :, :] = (
                comm_ref[recv_slot, :, :] + x_ref[pl.ds(r * chunk, chunk), :]
            )
        reduced = (my + 1) % N_DEV
        out_ref[pl.ds(reduced * chunk, chunk), :] = comm_ref[1, :, :]

        for t in range(N_DEV - 1):
            send_slot = (t + 1) % 2
            recv_slot = t % 2
            rdma = pltpu.make_async_remote_copy(
                src_ref=comm_ref.at[send_slot],
                dst_ref=comm_ref.at[recv_slot],
                send_sem=send_sems.at[send_slot],
                recv_sem=recv_sems.at[recv_slot],
                device_id=(right,),
                device_id_type=pl.DeviceIdType.MESH,
            )
            rdma.start()
            rdma.wait()
            g = (my - t) % N_DEV
            out_ref[pl.ds(g * chunk, chunk), :] = comm_ref[recv_slot, :, :]

    return pl.pallas_call(
        body,
        out_shape=jax.ShapeDtypeStruct((m, n), x.dtype),
        in_specs=[pl.BlockSpec(memory_space=pltpu.VMEM)],
        out_specs=pl.BlockSpec(memory_space=pltpu.VMEM),
        scratch_shapes=[
            pltpu.VMEM((2, chunk, n), x.dtype),
            pltpu.SemaphoreType.DMA((2,)),
            pltpu.SemaphoreType.DMA((2,)),
        ],
        compiler_params=pltpu.CompilerParams(collective_id=0),
    )(x)


# device time: 50458 ns/iter; 2.0429x vs baseline; 2.0429x over previous
import jax
import jax.numpy as jnp
from jax import lax
from jax.experimental import pallas as pl
from jax.experimental.pallas import tpu as pltpu

N_DEV = 16
N_STEPS = 4
DIMS = ("x", "y", "z1", "z2")

GROUPS = (
    (0, 512, ("x", "y", "z1", "z2")),
    (512, 512, ("z1", "z2", "x", "y")),
)


def kernel(x):
    m, n = x.shape
    n_g = len(GROUPS)

    def body(x_ref, out_ref, *scratch):
        comm = scratch[: n_g * N_STEPS]
        send_sems, recv_sems = scratch[n_g * N_STEPS :]

        my = lax.axis_index("i")
        p = my % 4
        z = my // 4
        coord = {
            "x": (p ^ (p >> 1)) & 1,
            "y": p >> 1,
            "z1": z & 1,
            "z2": (z >> 1) & 1,
        }
        partner = {
            "x": 4 * z + (p ^ 1),
            "y": 4 * z + (p ^ 3),
            "z1": 4 * (z ^ 1) + p,
            "z2": 4 * (z ^ 2) + p,
        }

        barrier_sem = pltpu.get_barrier_semaphore()
        for d in DIMS:
            pl.semaphore_signal(
                barrier_sem, inc=1,
                device_id=(partner[d],), device_id_type=pl.DeviceIdType.MESH,
            )
        pl.semaphore_wait(barrier_sem, 4)

        offs = [g0 for (g0, _, _) in GROUPS]
        for k in range(N_STEPS):
            rdmas, keeps = [], []
            for gi, (g0, r, order) in enumerate(GROUPS):
                d = order[k]
                h = r >> (k + 1)
                bit = coord[d]
                keep_off = offs[gi] + bit * h
                send_off = offs[gi] + (1 - bit) * h
                src = x_ref if k == 0 else out_ref
                rdma = pltpu.make_async_remote_copy(
                    src_ref=src.at[pl.ds(send_off, h)],
                    dst_ref=comm[gi * N_STEPS + k],
                    send_sem=send_sems.at[gi, k],
                    recv_sem=recv_sems.at[gi, k],
                    device_id=(partner[d],),
                    device_id_type=pl.DeviceIdType.MESH,
                )
                rdma.start()
                rdmas.append(rdma)
                keeps.append((keep_off, h))
            for gi in range(n_g):
                rdmas[gi].wait()
            for gi, (g0, r, order) in enumerate(GROUPS):
                keep_off, h = keeps[gi]
                src = x_ref if k == 0 else out_ref
                out_ref[pl.ds(keep_off, h), :] = (
                    src[pl.ds(keep_off, h), :] + comm[gi * N_STEPS + k][:, :]
                )
                offs[gi] = keep_off

        sizes = [r >> N_STEPS for (_, r, _) in GROUPS]
        for k in range(N_STEPS):
            rdmas = []
            for gi, (g0, r, order) in enumerate(GROUPS):
                d = order[N_STEPS - 1 - k]
                s = sizes[gi]
                rdma = pltpu.make_async_remote_copy(
                    src_ref=out_ref.at[pl.ds(offs[gi], s)],
                    dst_ref=out_ref.at[pl.ds(offs[gi], s)],
                    send_sem=send_sems.at[gi, N_STEPS + k],
                    recv_sem=recv_sems.at[gi, N_STEPS + k],
                    device_id=(partner[d],),
                    device_id_type=pl.DeviceIdType.MESH,
                )
                rdma.start()
                rdmas.append(rdma)
            for gi, (g0, r, order) in enumerate(GROUPS):
                rdmas[gi].wait()
                d = order[N_STEPS - 1 - k]
                offs[gi] = offs[gi] - coord[d] * sizes[gi]
                sizes[gi] = sizes[gi] * 2

    comm_shapes = [
        pltpu.VMEM((r >> (k + 1), n), x.dtype)
        for (_, r, _) in GROUPS
        for k in range(N_STEPS)
    ]
    return pl.pallas_call(
        body,
        out_shape=jax.ShapeDtypeStruct((m, n), x.dtype),
        in_specs=[pl.BlockSpec(memory_space=pltpu.VMEM)],
        out_specs=pl.BlockSpec(memory_space=pltpu.VMEM),
        scratch_shapes=comm_shapes + [
            pltpu.SemaphoreType.DMA((n_g, 2 * N_STEPS)),
            pltpu.SemaphoreType.DMA((n_g, 2 * N_STEPS)),
        ],
        compiler_params=pltpu.CompilerParams(collective_id=0),
    )(x)
